# baseline (speedup 1.0000x reference)
import functools

import jax
import jax.numpy as jnp
from jax.experimental import pallas as pl
from jax.experimental.pallas import tpu as pltpu

_GELU_C0 = 0.044715
_GELU_C1 = 0.7978845608028654


def _gelu_tanh(v):
    return (v * 0.5) * (1.0 + jnp.tanh(_GELU_C1 * (v + _GELU_C0 * v * v * v)))


def _fused_kernel(x_ref, w_ref, b_ref, o_ref, g_s, *, d_size, h, w, cout):
    dhw = x_ref.shape[1]
    c4 = w_ref.shape[1]

    v = jnp.dot(x_ref[0], w_ref[...], preferred_element_type=jnp.float32)
    v = v + jnp.broadcast_to(b_ref[...], (dhw, c4))
    g_s[...] = _gelu_tanh(v)

    for kh in range(2):
        for kw in range(2):
            t = 2 * kh + kw
            val = g_s[:, t * cout:(t + 1) * cout].reshape(d_size, h, w, cout)
            o_ref[0, pl.ds(0, d_size, 2), pl.ds(kh, h, 2), pl.ds(kw, w, 2), :] = val

    fill = jnp.broadcast_to(
        _gelu_tanh(b_ref[0:1, 0:cout]).reshape(1, 1, cout), (2 * h, 2 * w, cout))
    for d in range(d_size):
        o_ref[0, 2 * d + 1] = fill


def kernel(x, weight, bias):
    n, cin, d_size, h, w = x.shape
    cout = weight.shape[1]
    do, ho, wo = 2 * d_size - 1, 2 * h, 2 * w
    dhw = d_size * h * w
    c4 = 4 * cout

    xt = jnp.transpose(x, (0, 2, 3, 4, 1)).reshape(n, dhw, cin)
    w4t = jnp.transpose(weight[:, :, 0, :, :], (0, 2, 3, 1)).reshape(cin, c4)
    b4 = jnp.tile(bias, 4).reshape(1, c4)

    splits = 2
    d_blk = d_size // splits
    out5 = pl.pallas_call(
        functools.partial(_fused_kernel, d_size=d_blk, h=h, w=w, cout=cout),
        out_shape=jax.ShapeDtypeStruct((n, do, ho, wo, cout), jnp.float32),
        grid=(n, splits),
        in_specs=[
            pl.BlockSpec((1, dhw // splits, cin), lambda i, j: (i, j, 0)),
            pl.BlockSpec((cin, c4), lambda i, j: (0, 0)),
            pl.BlockSpec((1, c4), lambda i, j: (0, 0)),
        ],
        out_specs=pl.BlockSpec(
            (1, 2 * d_blk, ho, wo, cout), lambda i, j: (i, j, 0, 0, 0)),
        scratch_shapes=[pltpu.VMEM((dhw // splits, c4), jnp.float32)],
        compiler_params=pltpu.CompilerParams(
            dimension_semantics=("parallel", "arbitrary")),
    )(xt, w4t, b4)
    return jnp.transpose(out5, (0, 4, 1, 2, 3))


# speedup vs baseline: 1.1575x; 1.1575x over previous
import functools

import jax
import jax.numpy as jnp
from jax.experimental import pallas as pl
from jax.experimental.pallas import tpu as pltpu

_GELU_C0 = 0.044715
_GELU_C1 = 0.7978845608028654


def _gelu_tanh(v):
    return (v * 0.5) * (1.0 + jnp.tanh(_GELU_C1 * (v + _GELU_C0 * v * v * v)))


def _fused_kernel(x_ref, w_ref, b_ref, o_ref, g_s, *, d_size, h, w, cout):
    dhw = x_ref.shape[1]
    c4 = w_ref.shape[1]

    v = jnp.dot(x_ref[0], w_ref[...], preferred_element_type=jnp.float32)
    v = v + jnp.broadcast_to(b_ref[...], (dhw, c4))
    g_s[...] = _gelu_tanh(v)

    for kh in range(2):
        for kw in range(2):
            t = 2 * kh + kw
            val = g_s[:, t * cout:(t + 1) * cout].reshape(d_size, h, w, cout)
            o_ref[0, pl.ds(0, d_size, 2), pl.ds(kh, h, 2), pl.ds(kw, w, 2), :] = val

    fill = jnp.broadcast_to(
        _gelu_tanh(b_ref[0:1, 0:cout]).reshape(1, 1, cout), (2 * h, 2 * w, cout))
    for d in range(d_size - 1):
        o_ref[0, 2 * d + 1] = fill


def kernel(x, weight, bias):
    n, cin, d_size, h, w = x.shape
    cout = weight.shape[1]
    do, ho, wo = 2 * d_size - 1, 2 * h, 2 * w
    dhw = d_size * h * w
    c4 = 4 * cout

    xt = jnp.transpose(x, (0, 2, 3, 4, 1)).reshape(n, dhw, cin)
    w4t = jnp.transpose(weight[:, :, 0, :, :], (0, 2, 3, 1)).reshape(cin, c4)
    b4 = jnp.tile(bias, 4).reshape(1, c4)

    out5 = pl.pallas_call(
        functools.partial(_fused_kernel, d_size=d_size, h=h, w=w, cout=cout),
        out_shape=jax.ShapeDtypeStruct((n, do, ho, wo, cout), jnp.float32),
        grid=(n,),
        in_specs=[
            pl.BlockSpec((1, dhw, cin), lambda i: (i, 0, 0)),
            pl.BlockSpec((cin, c4), lambda i: (0, 0)),
            pl.BlockSpec((1, c4), lambda i: (0, 0)),
        ],
        out_specs=pl.BlockSpec((1, do, ho, wo, cout), lambda i: (i, 0, 0, 0, 0)),
        scratch_shapes=[pltpu.VMEM((dhw, c4), jnp.float32)],
        compiler_params=pltpu.CompilerParams(
            dimension_semantics=("parallel",)),
    )(xt, w4t, b4)
    return jnp.transpose(out5, (0, 4, 1, 2, 3))
